# baseline (speedup 1.0000x reference)
import functools

import jax
import jax.numpy as jnp
from jax import lax
from jax.experimental import pallas as pl
from jax.experimental.pallas import tpu as pltpu
from jax.experimental.pallas import tpu_sc as plsc

_R = 64
_B = 128
_N = 10000
_K = 20

_NB = 400
_NBLK = _N // _NB

_L = 16
_NW = 32
_CH = 313
_CH_LAST = _N - 31 * _CH
_ROWS = 320
_SG = 20
_NGRP = _B // _L
_NC = _NW * _K
_SG2 = _NC // _L

_NEG = float("-inf")
_BIG = 2**30


def _score_body(w_ref, gx_ref, bias_ref, out_ref):
    xb = gx_ref[...].astype(jnp.bfloat16)
    x2 = xb.reshape(_R, _NB * _B)
    wb = w_ref[...]
    acc = lax.dot_general(
        wb, x2, (((1,), (0,)), ((), ())),
        preferred_element_type=jnp.float32,
    ).reshape(_NB, _B)
    out_ref[...] = acc + bias_ref[...]


def _score(gx_t, wb, bias2):
    return pl.pallas_call(
        _score_body,
        grid=(_NBLK,),
        in_specs=[
            pl.BlockSpec((1, _R), lambda j: (0, 0)),
            pl.BlockSpec((_R, _NB, _B), lambda j: (0, j, 0)),
            pl.BlockSpec((_NB, 1), lambda j: (j, 0)),
        ],
        out_specs=pl.BlockSpec((_NB, _B), lambda j: (j, 0)),
        out_shape=jax.ShapeDtypeStruct((_N, _B), jnp.float32),
        compiler_params=pltpu.CompilerParams(
            dimension_semantics=("arbitrary",),
        ),
    )(wb, gx_t, bias2)


def _neg16():
    return jnp.full((_L,), _NEG, jnp.float32)


def _build_gm(data_v, gm_v, sl, n_sg):
    def sg_body(sg, _):
        acc = _neg16()
        for j in range(_L):
            acc = jnp.maximum(acc, data_v[sg * _L + j, sl])
        gm_v[sg, sl] = acc
        return 0
    lax.fori_loop(0, n_sg, sg_body, 0)


def _select_topk(data_v, gm_v, sl, lane_idx, n_sg, emit):
    def k_body(kk, _):
        def m_body(sg, m):
            return jnp.maximum(m, gm_v[sg, sl])
        m = lax.fori_loop(0, n_sg, m_body, _neg16())

        def a_body(sg, a):
            v = gm_v[sg, sl]
            hit = (v == m) & (a == _BIG)
            return jnp.where(hit, sg, a)
        argsg = lax.fori_loop(0, n_sg, a_body, jnp.full((_L,), _BIG, jnp.int32))

        base = argsg * _L
        rbest = jnp.full((_L,), _BIG, jnp.int32)
        vals = []
        for j in range(_L):
            row = base + j
            vj = plsc.load_gather(data_v, [row, lane_idx])
            vals.append((row, vj))
            hit = (vj == m) & (rbest == _BIG)
            rbest = jnp.where(hit, row, rbest)
        newgm = _neg16()
        for row, vj in vals:
            vj2 = jnp.where(row == rbest, _NEG, vj)
            newgm = jnp.maximum(newgm, vj2)
        plsc.store_scatter(data_v, [rbest, lane_idx], _neg16())
        plsc.store_scatter(gm_v, [argsg, lane_idx], newgm)
        emit(kk, m, rbest)
        return 0

    lax.fori_loop(0, _K, k_body, 0)


def _make_stage1():
    mesh = plsc.VectorSubcoreMesh(core_axis_name="c", subcore_axis_name="s")

    @functools.partial(
        pl.kernel, mesh=mesh,
        compiler_params=pltpu.CompilerParams(use_tc_tiling_on_sc=False, needs_layout_passes=False),
        out_type=[
            jax.ShapeDtypeStruct((_NC, _B), jnp.float32),
            jax.ShapeDtypeStruct((_NC, _B), jnp.int32),
        ],
        scratch_types=[
            pltpu.VMEM((_ROWS, _B), jnp.float32),
            pltpu.VMEM((_SG, _B), jnp.float32),
            pltpu.VMEM((_K, _B), jnp.float32),
            pltpu.VMEM((_K, _B), jnp.int32),
        ],
    )
    def stage1(score_hbm, cval_hbm, cidx_hbm, data_v, gm_v, cv_v, ci_v):
        w = lax.axis_index("c") * 16 + lax.axis_index("s")
        n0 = w * _CH

        for g in range(_NGRP):
            sl = pl.ds(g * _L, _L)
            for row in range(_CH, _ROWS):
                data_v[row, sl] = _neg16()

        @pl.when(w < _NW - 1)
        def _():
            pltpu.sync_copy(score_hbm.at[pl.ds(n0, _CH)],
                            data_v.at[pl.ds(0, _CH)])

        @pl.when(w == _NW - 1)
        def _():
            for g in range(_NGRP):
                sl = pl.ds(g * _L, _L)
                for row in range(_CH_LAST, _CH):
                    data_v[row, sl] = _neg16()
            pltpu.sync_copy(score_hbm.at[pl.ds(31 * _CH, _CH_LAST)],
                            data_v.at[pl.ds(0, _CH_LAST)])

        for g in range(_NGRP):
            sl = pl.ds(g * _L, _L)
            lane_idx = lax.iota(jnp.int32, _L) + g * _L
            _build_gm(data_v, gm_v, sl, _SG)

            def emit(kk, m, rbest, _sl=sl):
                cv_v[kk, _sl] = m
                ci_v[kk, _sl] = n0 + rbest

            _select_topk(data_v, gm_v, sl, lane_idx, _SG, emit)

        pltpu.sync_copy(cv_v, cval_hbm.at[pl.ds(w * _K, _K)])
        pltpu.sync_copy(ci_v, cidx_hbm.at[pl.ds(w * _K, _K)])

    return stage1


def _make_stage2():
    mesh = plsc.VectorSubcoreMesh(core_axis_name="c", subcore_axis_name="s")

    @functools.partial(
        pl.kernel, mesh=mesh,
        compiler_params=pltpu.CompilerParams(use_tc_tiling_on_sc=False, needs_layout_passes=False),
        out_type=[
            jax.ShapeDtypeStruct((_K, _B), jnp.float32),
            jax.ShapeDtypeStruct((_K, _B), jnp.int32),
        ],
        scratch_types=[
            pltpu.VMEM((_NC, _L), jnp.float32),
            pltpu.VMEM((_NC, _L), jnp.int32),
            pltpu.VMEM((_SG2, _L), jnp.float32),
            pltpu.VMEM((_K, _L), jnp.float32),
            pltpu.VMEM((_K, _L), jnp.int32),
        ],
    )
    def stage2(cval_hbm, cidx_hbm, tv_hbm, ti_hbm, v2_v, i2_v, gm_v,
               ov_v, oi_v):
        w = lax.axis_index("c") * 16 + lax.axis_index("s")

        @pl.when(w < _NGRP)
        def _():
            sl = pl.ds(w * _L, _L)
            pltpu.sync_copy(cval_hbm.at[:, sl], v2_v)
            pltpu.sync_copy(cidx_hbm.at[:, sl], i2_v)
            lane_idx = lax.iota(jnp.int32, _L)
            sl2 = pl.ds(0, _L)
            _build_gm(v2_v, gm_v, sl2, _SG2)

            def emit(kk, m, rbest):
                ov_v[kk, sl2] = m
                oi_v[kk, sl2] = plsc.load_gather(i2_v, [rbest, lane_idx])

            _select_topk(v2_v, gm_v, sl2, lane_idx, _SG2, emit)
            pltpu.sync_copy(ov_v, tv_hbm.at[:, sl])
            pltpu.sync_copy(oi_v, ti_hbm.at[:, sl])

    return stage2


def _topk_sc(score_t):
    cval, cidx = _make_stage1()(score_t)
    return _make_stage2()(cval, cidx)


def kernel(grounding_x, rule_weights, bias, all_h, all_r, k):
    gx_t = jnp.transpose(grounding_x, (0, 2, 1))
    wb = rule_weights.astype(jnp.bfloat16).reshape(1, _R)
    bias2 = bias.reshape(_N, 1)
    score_t = _score(gx_t, wb, bias2)
    vals_t, idx_t = _topk_sc(score_t)
    mask = jnp.ones((_B, _N), dtype=jnp.bool_)
    return score_t.T, mask, vals_t.T, idx_t.T


# speedup vs baseline: 1.1833x; 1.1833x over previous
import functools

import jax
import jax.numpy as jnp
from jax import lax
from jax.experimental import pallas as pl
from jax.experimental.pallas import tpu as pltpu
from jax.experimental.pallas import tpu_sc as plsc

_R = 64
_B = 128
_N = 10000
_K = 20

_NB = 400
_NBLK = _N // _NB

_L = 16
_KP = 24
_NC = _NBLK * _KP
_ROWS2 = 608
_SG2 = _ROWS2 // _L
_NGRP = _B // _L

_NEG = float("-inf")
_BIG = 2**30


def _score_body(w_ref, gx_ref, bias_ref, out_ref, cval_ref, cidx_ref):
    xb = gx_ref[...].astype(jnp.bfloat16)
    x2 = xb.reshape(_R, _NB * _B)
    wb = w_ref[...]
    acc = lax.dot_general(
        wb, x2, (((1,), (0,)), ((), ())),
        preferred_element_type=jnp.float32,
    ).reshape(_NB, _B)
    s = acc + bias_ref[...]
    out_ref[...] = s

    j = pl.program_id(0)
    row = lax.broadcasted_iota(jnp.int32, (_NB, _B), 0)
    krow = lax.broadcasted_iota(jnp.int32, (_KP, _B), 0)
    vals = jnp.full((_KP, _B), _NEG, jnp.float32)
    idxs = jnp.zeros((_KP, _B), jnp.int32)
    for i in range(_K):
        m = jnp.max(s, axis=0, keepdims=True)
        cand = jnp.where(s == m, row, jnp.int32(_BIG))
        ix = jnp.min(cand, axis=0, keepdims=True)
        vals = jnp.where(krow == i, m, vals)
        idxs = jnp.where(krow == i, ix + j * _NB, idxs)
        s = jnp.where(row == ix, _NEG, s)
    cval_ref[...] = vals
    cidx_ref[...] = idxs


def _score(gx_t, wb, bias2):
    return pl.pallas_call(
        _score_body,
        grid=(_NBLK,),
        in_specs=[
            pl.BlockSpec((1, _R), lambda j: (0, 0)),
            pl.BlockSpec((_R, _NB, _B), lambda j: (0, j, 0)),
            pl.BlockSpec((_NB, 1), lambda j: (j, 0)),
        ],
        out_specs=[
            pl.BlockSpec((_NB, _B), lambda j: (j, 0)),
            pl.BlockSpec((_KP, _B), lambda j: (j, 0)),
            pl.BlockSpec((_KP, _B), lambda j: (j, 0)),
        ],
        out_shape=[
            jax.ShapeDtypeStruct((_N, _B), jnp.float32),
            jax.ShapeDtypeStruct((_NC, _B), jnp.float32),
            jax.ShapeDtypeStruct((_NC, _B), jnp.int32),
        ],
        compiler_params=pltpu.CompilerParams(
            dimension_semantics=("arbitrary",),
        ),
    )(wb, gx_t, bias2)


def _neg16():
    return jnp.full((_L,), _NEG, jnp.float32)


def _make_merge():
    mesh = plsc.VectorSubcoreMesh(core_axis_name="c", subcore_axis_name="s")

    @functools.partial(
        pl.kernel, mesh=mesh,
        compiler_params=pltpu.CompilerParams(
            use_tc_tiling_on_sc=False, needs_layout_passes=False),
        out_type=[
            jax.ShapeDtypeStruct((_K, _B), jnp.float32),
            jax.ShapeDtypeStruct((_K, _B), jnp.int32),
        ],
        scratch_types=[
            pltpu.VMEM((_ROWS2, _L), jnp.float32),
            pltpu.VMEM((_NC, _L), jnp.int32),
            pltpu.VMEM((_SG2, _L), jnp.float32),
            pltpu.VMEM((_K, _L), jnp.float32),
            pltpu.VMEM((_K, _L), jnp.int32),
        ],
    )
    def merge(cval_hbm, cidx_hbm, tv_hbm, ti_hbm, v2_v, i2_v, gm_v,
              ov_v, oi_v):
        w = lax.axis_index("c") * 16 + lax.axis_index("s")

        @pl.when(w < _NGRP)
        def _():
            sl = pl.ds(w * _L, _L)
            sl2 = pl.ds(0, _L)
            for r in range(_NC, _ROWS2):
                v2_v[r, sl2] = _neg16()
            pltpu.sync_copy(cval_hbm.at[:, sl], v2_v.at[pl.ds(0, _NC)])
            pltpu.sync_copy(cidx_hbm.at[:, sl], i2_v)
            lane_idx = lax.iota(jnp.int32, _L)

            def sg_body(sg, _):
                acc = _neg16()
                for jj in range(_L):
                    acc = jnp.maximum(acc, v2_v[sg * _L + jj, sl2])
                gm_v[sg, sl2] = acc
                return 0
            lax.fori_loop(0, _SG2, sg_body, 0)

            def k_body(kk, _):
                def m_body(sg, m):
                    return jnp.maximum(m, gm_v[sg, sl2])
                m = lax.fori_loop(0, _SG2, m_body, _neg16())

                def a_body(sg, a):
                    v = gm_v[sg, sl2]
                    hit = (v == m) & (a == _BIG)
                    return jnp.where(hit, sg, a)
                argsg = lax.fori_loop(0, _SG2, a_body,
                                      jnp.full((_L,), _BIG, jnp.int32))

                base = argsg * _L
                rbest = jnp.full((_L,), _BIG, jnp.int32)
                vals = []
                for jj in range(_L):
                    rr = base + jj
                    vj = plsc.load_gather(v2_v, [rr, lane_idx])
                    vals.append((rr, vj))
                    hit = (vj == m) & (rbest == _BIG)
                    rbest = jnp.where(hit, rr, rbest)
                newgm = _neg16()
                for rr, vj in vals:
                    vj2 = jnp.where(rr == rbest, _NEG, vj)
                    newgm = jnp.maximum(newgm, vj2)
                plsc.store_scatter(v2_v, [rbest, lane_idx], _neg16())
                plsc.store_scatter(gm_v, [argsg, lane_idx], newgm)
                ov_v[kk, sl2] = m
                oi_v[kk, sl2] = plsc.load_gather(i2_v, [rbest, lane_idx])
                return 0

            lax.fori_loop(0, _K, k_body, 0)
            pltpu.sync_copy(ov_v, tv_hbm.at[:, sl])
            pltpu.sync_copy(oi_v, ti_hbm.at[:, sl])

    return merge


def kernel(grounding_x, rule_weights, bias, all_h, all_r, k):
    gx_t = jnp.transpose(grounding_x, (0, 2, 1))
    wb = rule_weights.astype(jnp.bfloat16).reshape(1, _R)
    bias2 = bias.reshape(_N, 1)
    score_t, cval, cidx = _score(gx_t, wb, bias2)
    vals_t, idx_t = _make_merge()(cval, cidx)
    mask = jnp.ones((_B, _N), dtype=jnp.bool_)
    return score_t.T, mask, vals_t.T, idx_t.T


# speedup vs baseline: 1.2085x; 1.0212x over previous
import functools

import jax
import jax.numpy as jnp
from jax import lax
from jax.experimental import pallas as pl
from jax.experimental.pallas import tpu as pltpu
from jax.experimental.pallas import tpu_sc as plsc

_R = 64
_B = 128
_N = 10000
_K = 20

_NB = 400
_NBLK = _N // _NB

_L = 16
_KP = 24
_NC = _NBLK * _KP
_ROWS2 = 608
_SG2 = _ROWS2 // _L
_NGRP = _B // _L

_NEG = float("-inf")
_BIG = 2**30


def _score_body(w_ref, gx_ref, bias_ref, out_ref, cval_ref, cidx_ref):
    xb = gx_ref[...].astype(jnp.bfloat16)
    x2 = xb.reshape(_R, _NB * _B)
    wb = w_ref[...]
    acc = lax.dot_general(
        wb, x2, (((1,), (0,)), ((), ())),
        preferred_element_type=jnp.float32,
    ).reshape(_NB, _B)
    s = acc + bias_ref[...]
    out_ref[...] = s

    j = pl.program_id(0)
    row = lax.broadcasted_iota(jnp.int32, (_NB, _B), 0)
    krow = lax.broadcasted_iota(jnp.int32, (_KP, _B), 0)
    vals = jnp.full((_KP, _B), _NEG, jnp.float32)
    idxs = jnp.zeros((_KP, _B), jnp.int32)
    for i in range(_K):
        m = jnp.max(s, axis=0, keepdims=True)
        cand = jnp.where(s == m, row, jnp.int32(_BIG))
        ix = jnp.min(cand, axis=0, keepdims=True)
        vals = jnp.where(krow == i, m, vals)
        idxs = jnp.where(krow == i, ix + j * _NB, idxs)
        s = jnp.where(row == ix, _NEG, s)
    cval_ref[...] = vals
    cidx_ref[...] = idxs


def _score(gx_t, wb, bias2):
    return pl.pallas_call(
        _score_body,
        grid=(_NBLK,),
        in_specs=[
            pl.BlockSpec((1, _R), lambda j: (0, 0)),
            pl.BlockSpec((_R, _NB, _B), lambda j: (0, j, 0)),
            pl.BlockSpec((_NB, 1), lambda j: (j, 0)),
        ],
        out_specs=[
            pl.BlockSpec((_NB, _B), lambda j: (j, 0)),
            pl.BlockSpec((_KP, _B), lambda j: (j, 0)),
            pl.BlockSpec((_KP, _B), lambda j: (j, 0)),
        ],
        out_shape=[
            jax.ShapeDtypeStruct((_N, _B), jnp.float32),
            jax.ShapeDtypeStruct((_NC, _B), jnp.float32),
            jax.ShapeDtypeStruct((_NC, _B), jnp.int32),
        ],
        compiler_params=pltpu.CompilerParams(
            dimension_semantics=("arbitrary",),
        ),
    )(wb, gx_t, bias2)


def _neg16():
    return jnp.full((_L,), _NEG, jnp.float32)


def _make_merge():
    mesh = plsc.VectorSubcoreMesh(core_axis_name="c", subcore_axis_name="s")

    @functools.partial(
        pl.kernel, mesh=mesh,
        compiler_params=pltpu.CompilerParams(
            use_tc_tiling_on_sc=False, needs_layout_passes=False),
        out_type=[
            jax.ShapeDtypeStruct((_K, _B), jnp.float32),
            jax.ShapeDtypeStruct((_K, _B), jnp.int32),
        ],
        scratch_types=[
            pltpu.VMEM((_ROWS2, _L), jnp.float32),
            pltpu.VMEM((_NC, _L), jnp.int32),
            pltpu.VMEM((_SG2, _L), jnp.float32),
            pltpu.VMEM((_K, _L), jnp.float32),
            pltpu.VMEM((_K, _L), jnp.int32),
        ],
    )
    def merge(cval_hbm, cidx_hbm, tv_hbm, ti_hbm, v2_v, i2_v, gm_v,
              ov_v, oi_v):
        w = lax.axis_index("c") * 16 + lax.axis_index("s")

        @pl.when(w < _NGRP)
        def _():
            sl = pl.ds(w * _L, _L)
            sl2 = pl.ds(0, _L)
            for r in range(_NC, _ROWS2):
                v2_v[r, sl2] = _neg16()
            pltpu.sync_copy(cval_hbm.at[:, sl], v2_v.at[pl.ds(0, _NC)])
            pltpu.sync_copy(cidx_hbm.at[:, sl], i2_v)
            lane_idx = lax.iota(jnp.int32, _L)

            def sg_body(sg, _):
                acc = _neg16()
                for jj in range(_L):
                    acc = jnp.maximum(acc, v2_v[sg * _L + jj, sl2])
                gm_v[sg, sl2] = acc
                return 0
            lax.fori_loop(0, _SG2, sg_body, 0)

            def k_body(kk, _):
                def ma_body(sg, carry):
                    m, a = carry
                    v = gm_v[sg, sl2]
                    gt = v > m
                    return jnp.where(gt, v, m), jnp.where(gt, sg, a)
                m, argsg = lax.fori_loop(
                    0, _SG2, ma_body,
                    (_neg16(), jnp.full((_L,), _BIG, jnp.int32)))

                base = argsg * _L
                rbest = jnp.full((_L,), _BIG, jnp.int32)
                vals = []
                for jj in range(_L):
                    rr = base + jj
                    vj = plsc.load_gather(v2_v, [rr, lane_idx])
                    vals.append((rr, vj))
                    hit = (vj == m) & (rbest == _BIG)
                    rbest = jnp.where(hit, rr, rbest)
                newgm = _neg16()
                for rr, vj in vals:
                    vj2 = jnp.where(rr == rbest, _NEG, vj)
                    newgm = jnp.maximum(newgm, vj2)
                plsc.store_scatter(v2_v, [rbest, lane_idx], _neg16())
                plsc.store_scatter(gm_v, [argsg, lane_idx], newgm)
                ov_v[kk, sl2] = m
                oi_v[kk, sl2] = plsc.load_gather(i2_v, [rbest, lane_idx])
                return 0

            lax.fori_loop(0, _K, k_body, 0)
            pltpu.sync_copy(ov_v, tv_hbm.at[:, sl])
            pltpu.sync_copy(oi_v, ti_hbm.at[:, sl])

    return merge


def kernel(grounding_x, rule_weights, bias, all_h, all_r, k):
    gx_t = jnp.transpose(grounding_x, (0, 2, 1))
    wb = rule_weights.astype(jnp.bfloat16).reshape(1, _R)
    bias2 = bias.reshape(_N, 1)
    score_t, cval, cidx = _score(gx_t, wb, bias2)
    vals_t, idx_t = _make_merge()(cval, cidx)
    mask = jnp.ones((_B, _N), dtype=jnp.bool_)
    return score_t.T, mask, vals_t.T, idx_t.T
